# baseline (speedup 1.0000x reference)
import functools

import jax
import jax.numpy as jnp
from jax import lax
from jax.experimental import pallas as pl
from jax.experimental.pallas import tpu as pltpu
from jax.experimental.pallas import tpu_sc as plsc

D_MODEL = 2048
NC = 2
NS = 16
NW = NC * NS
B = 4 * 8192
B_PER_W = B // NW
CHUNK = 16
NCH = B_PER_W // CHUNK
NBUF = 3
MAIN = ((NCH - 3 - 1) // NBUF) * NBUF

_mesh = plsc.VectorSubcoreMesh(
    core_axis_name="c", subcore_axis_name="s", num_cores=NC, num_subcores=NS
)


@functools.partial(
    pl.kernel,
    mesh=_mesh,
    out_type=jax.ShapeDtypeStruct((B, D_MODEL), jnp.float32),
    scratch_types=[
        pltpu.VMEM((B_PER_W,), jnp.int32),
        pltpu.VMEM((CHUNK, D_MODEL), jnp.float32),
        pltpu.VMEM((CHUNK, D_MODEL), jnp.float32),
        pltpu.VMEM((CHUNK, D_MODEL), jnp.float32),
        pltpu.SemaphoreType.DMA,
        pltpu.SemaphoreType.DMA,
        pltpu.SemaphoreType.DMA,
        pltpu.SemaphoreType.DMA,
        pltpu.SemaphoreType.DMA,
        pltpu.SemaphoreType.DMA,
    ],
)
def _gather_rows(
    table_hbm, idx_hbm, out_hbm, idx_v, r0, r1, r2, g0, g1, g2, w0, w1, w2
):
    rows = (r0, r1, r2)
    gsems = (g0, g1, g2)
    wsems = (w0, w1, w2)
    wid = lax.axis_index("s") * NC + lax.axis_index("c")
    base = wid * B_PER_W
    pltpu.sync_copy(idx_hbm.at[pl.ds(base, B_PER_W)], idx_v)

    def gather_start(off, b):
        pltpu.async_copy(
            table_hbm.at[idx_v.at[pl.ds(off, CHUNK)]], rows[b], gsems[b]
        )

    def gather_wait(b):
        pltpu.make_async_copy(
            table_hbm.at[pl.ds(0, CHUNK)], rows[b], gsems[b]
        ).wait()

    def write_start(off, b):
        pltpu.async_copy(
            rows[b], out_hbm.at[pl.ds(base + off, CHUNK)], wsems[b]
        )

    def write_wait(b):
        pltpu.make_async_copy(
            out_hbm.at[pl.ds(0, CHUNK)], rows[b], wsems[b]
        ).wait()

    gather_start(0, 0)
    gather_start(1 * CHUNK, 1)
    gather_wait(0)
    write_start(0, 0)
    gather_start(2 * CHUNK, 2)

    @pl.loop(1, 1 + MAIN, step=NBUF)
    def _(c):
        for u in range(NBUF):
            b = (1 + u) % NBUF
            gather_wait(b)
            write_start((c + u) * CHUNK, b)
            write_wait(u % NBUF)
            gather_start((c + u + 2) * CHUNK, u % NBUF)

    for i in range(1 + MAIN, NCH):
        b = i % NBUF
        gather_wait(b)
        write_start(i * CHUNK, b)
        if i + 2 < NCH:
            write_wait((i - 1) % NBUF)
            gather_start((i + 2) * CHUNK, (i + 2) % NBUF)

    for i in range(NCH - 3, NCH):
        write_wait(i % NBUF)


def kernel(positions, table):
    idx = positions.reshape(-1).astype(jnp.int32)
    out = _gather_rows(table, idx)
    return out.reshape(*positions.shape, D_MODEL)


# speedup vs baseline: 1.0045x; 1.0045x over previous
import functools

import jax
import jax.numpy as jnp
from jax import lax
from jax.experimental import pallas as pl
from jax.experimental.pallas import tpu as pltpu
from jax.experimental.pallas import tpu_sc as plsc

D_MODEL = 2048
NC = 2
NS = 16
NW = NC * NS
B = 4 * 8192
B_PER_W = B // NW
CHUNK = 8
NCH = B_PER_W // CHUNK
NBUF = 6
MAIN = ((NCH - NBUF - 1) // NBUF) * NBUF

_mesh = plsc.VectorSubcoreMesh(
    core_axis_name="c", subcore_axis_name="s", num_cores=NC, num_subcores=NS
)


@functools.partial(
    pl.kernel,
    mesh=_mesh,
    out_type=jax.ShapeDtypeStruct((B, D_MODEL), jnp.float32),
    scratch_types=[
        pltpu.VMEM((B_PER_W,), jnp.int32),
        [pltpu.VMEM((CHUNK, D_MODEL), jnp.float32) for _ in range(NBUF)],
        [pltpu.SemaphoreType.DMA for _ in range(NBUF)],
        [pltpu.SemaphoreType.DMA for _ in range(NBUF)],
    ],
)
def _gather_rows(table_hbm, idx_hbm, out_hbm, idx_v, rows, gsems, wsems):
    wid = lax.axis_index("s") * NC + lax.axis_index("c")
    base = wid * B_PER_W
    pltpu.sync_copy(idx_hbm.at[pl.ds(base, B_PER_W)], idx_v)

    def gather_start(c, b):
        pltpu.async_copy(
            table_hbm.at[idx_v.at[pl.ds(c * CHUNK, CHUNK)]], rows[b], gsems[b]
        )

    def gather_wait(b):
        pltpu.make_async_copy(
            table_hbm.at[pl.ds(0, CHUNK)], rows[b], gsems[b]
        ).wait()

    def write_start(c, b):
        pltpu.async_copy(
            rows[b], out_hbm.at[pl.ds(base + c * CHUNK, CHUNK)], wsems[b]
        )

    def write_wait(b):
        pltpu.make_async_copy(
            out_hbm.at[pl.ds(0, CHUNK)], rows[b], wsems[b]
        ).wait()

    for j in range(NBUF - 1):
        gather_start(j, j)
    gather_wait(0)
    write_start(0, 0)
    gather_start(NBUF - 1, NBUF - 1)

    @pl.loop(1, 1 + MAIN, step=NBUF)
    def _(c):
        for u in range(NBUF):
            b = (1 + u) % NBUF
            gather_wait(b)
            write_start(c + u, b)
            write_wait(u % NBUF)
            gather_start(c + u + NBUF - 1, u % NBUF)

    for i in range(1 + MAIN, NCH):
        b = i % NBUF
        gather_wait(b)
        write_start(i, b)
        if i + NBUF - 1 < NCH:
            write_wait((i - 1) % NBUF)
            gather_start(i + NBUF - 1, (i - 1) % NBUF)

    for i in range(NCH - NBUF, NCH):
        write_wait(i % NBUF)


def kernel(positions, table):
    idx = positions.reshape(-1).astype(jnp.int32)
    out = _gather_rows(table, idx)
    return out.reshape(*positions.shape, D_MODEL)


# speedup vs baseline: 1.0094x; 1.0049x over previous
import functools

import jax
import jax.numpy as jnp
from jax import lax
from jax.experimental import pallas as pl
from jax.experimental.pallas import tpu as pltpu
from jax.experimental.pallas import tpu_sc as plsc

D_MODEL = 2048
NC = 2
NS = 16
NW = NC * NS
B = 4 * 8192
B_PER_W = B // NW
CHUNK = 8
NCH = B_PER_W // CHUNK
NBUF = 7
MAIN = ((NCH - NBUF - 1) // NBUF) * NBUF

_mesh = plsc.VectorSubcoreMesh(
    core_axis_name="c", subcore_axis_name="s", num_cores=NC, num_subcores=NS
)


@functools.partial(
    pl.kernel,
    mesh=_mesh,
    out_type=jax.ShapeDtypeStruct((B, D_MODEL), jnp.float32),
    scratch_types=[
        pltpu.VMEM((B_PER_W,), jnp.int32),
        [pltpu.VMEM((CHUNK, D_MODEL), jnp.float32) for _ in range(NBUF)],
        [pltpu.SemaphoreType.DMA for _ in range(NBUF)],
        [pltpu.SemaphoreType.DMA for _ in range(NBUF)],
    ],
)
def _gather_rows(table_hbm, idx_hbm, out_hbm, idx_v, rows, gsems, wsems):
    wid = lax.axis_index("s") * NC + lax.axis_index("c")
    base = wid * B_PER_W
    pltpu.sync_copy(idx_hbm.at[pl.ds(base, B_PER_W)], idx_v)

    def gather_start(c, b):
        pltpu.async_copy(
            table_hbm.at[idx_v.at[pl.ds(c * CHUNK, CHUNK)]], rows[b], gsems[b]
        )

    def gather_wait(b):
        pltpu.make_async_copy(
            table_hbm.at[pl.ds(0, CHUNK)], rows[b], gsems[b]
        ).wait()

    def write_start(c, b):
        pltpu.async_copy(
            rows[b], out_hbm.at[pl.ds(base + c * CHUNK, CHUNK)], wsems[b]
        )

    def write_wait(b):
        pltpu.make_async_copy(
            out_hbm.at[pl.ds(0, CHUNK)], rows[b], wsems[b]
        ).wait()

    for j in range(NBUF - 1):
        gather_start(j, j)
    gather_wait(0)
    write_start(0, 0)
    gather_start(NBUF - 1, NBUF - 1)

    @pl.loop(1, 1 + MAIN, step=NBUF)
    def _(c):
        for u in range(NBUF):
            b = (1 + u) % NBUF
            gather_wait(b)
            write_start(c + u, b)
            write_wait(u % NBUF)
            gather_start(c + u + NBUF - 1, u % NBUF)

    for i in range(1 + MAIN, NCH):
        b = i % NBUF
        gather_wait(b)
        write_start(i, b)
        if i + NBUF - 1 < NCH:
            write_wait((i - 1) % NBUF)
            gather_start(i + NBUF - 1, (i - 1) % NBUF)

    for i in range(NCH - NBUF, NCH):
        write_wait(i % NBUF)


def kernel(positions, table):
    idx = positions.reshape(-1).astype(jnp.int32)
    out = _gather_rows(table, idx)
    return out.reshape(*positions.shape, D_MODEL)
